# Initial kernel scaffold; baseline (speedup 1.0000x reference)
import jax
import jax.numpy as jnp
from jax.experimental import pallas as pl


def kernel(x, edge_index, batch, w_pre1, b_pre1, w_pre2, b_pre2, w_pre3, b_pre3, w_pre4, b_pre4, gin0_w1, gin0_b1, gin0_bng, gin0_bnb, gin0_w2, gin0_b2, gin1_w1, gin1_b1, gin1_bng, gin1_bnb, gin1_w2, gin1_b2, bn0_g, bn0_b, bn1_g, bn1_b, w_post1, b_post1, w_post2, b_post2):
    raise NotImplementedError("write your pallas kernel here")


# speedup vs baseline: 5.4025x; 5.4025x over previous
import functools

import jax
import jax.numpy as jnp
from jax import lax
from jax.experimental import pallas as pl
from jax.experimental.pallas import tpu as pltpu
from jax.experimental.pallas import tpu_sc as plsc

N = 10000
E = 320000
D = 128
CFEA = 2
H = 64
NG = 64

BN = 1024
N_PAD = 10240
GRID = N_PAD // BN

NTILES = 32
EPT = 10240
E_PAD = NTILES * EPT
CHUNK = 128
NCH = EPT // CHUNK
NPAIR = NCH // 2
RPS = N_PAD // 16

_BN_INV = (1.0 + 1e-5) ** -0.5


@functools.cache
def _make_sc_agg():
    mesh = plsc.VectorSubcoreMesh(
        core_axis_name="c", subcore_axis_name="s", num_cores=2, num_subcores=16
    )
    return pl.kernel(
        _sc_agg_body,
        out_type=jax.ShapeDtypeStruct((2, N_PAD, H), jnp.float32),
        mesh=mesh,
        scratch_types=[
            pltpu.VMEM((NCH, CHUNK), jnp.int32),
            pltpu.VMEM((NCH, CHUNK), jnp.int32),
            pltpu.VMEM((2, CHUNK, H), jnp.float32),
            pltpu.VMEM_SHARED((N_PAD, H), jnp.float32),
            pltpu.SemaphoreType.DMA,
            pltpu.SemaphoreType.DMA,
        ],
        compiler_params=pltpu.CompilerParams(use_tc_tiling_on_sc=False),
    )


def _sc_agg(h, src3, dst3):
    return _make_sc_agg()(h, src3, dst3)


def _sc_agg_body(h_hbm, src_hbm, dst_hbm, out_hbm, src_v, dst_v, buf, acc, sem0, sem1):
    c = lax.axis_index("c")
    s = lax.axis_index("s")
    w = c * 16 + s

    pltpu.sync_copy(src_hbm.at[w], src_v)
    pltpu.sync_copy(dst_hbm.at[w], dst_v)

    def _zrow(i, carry):
        for k4 in range(4):
            buf[0, i, pl.ds(k4 * 16, 16)] = jnp.zeros((16,), jnp.float32)
        return carry

    lax.fori_loop(0, CHUNK, _zrow, 0)

    def _crow(j, carry):
        pltpu.sync_copy(buf.at[0], acc.at[pl.ds(s * RPS + j * CHUNK, CHUNK)])
        return carry

    lax.fori_loop(0, RPS // CHUNK, _crow, 0)
    plsc.subcore_barrier()

    pltpu.async_copy(h_hbm.at[src_v.at[0]], buf.at[0], sem0)

    def _pair(jj, carry):
        j0 = 2 * jj
        cp1 = pltpu.async_copy(h_hbm.at[src_v.at[j0 + 1]], buf.at[1], sem1)
        pltpu.make_async_copy(h_hbm.at[src_v.at[0]], buf.at[0], sem0).wait()
        pltpu.sync_copy(buf.at[0], acc.at[dst_v.at[j0]], add=True)

        @pl.when(jj < NPAIR - 1)
        def _():
            pltpu.async_copy(h_hbm.at[src_v.at[j0 + 2]], buf.at[0], sem0)

        cp1.wait()
        pltpu.sync_copy(buf.at[1], acc.at[dst_v.at[j0 + 1]], add=True)
        return carry

    lax.fori_loop(0, NPAIR, _pair, 0)
    plsc.subcore_barrier()

    r0 = s * RPS
    pltpu.sync_copy(acc.at[pl.ds(r0, RPS)], out_hbm.at[c, pl.ds(r0, RPS)])


def _pre_body(x_ref, wa_ref, ba_ref, wb_ref, bb_ref, wa2_ref, wb2_ref, b2_ref,
              o_ref):
    x = x_ref[...]
    ha = jnp.maximum(
        jnp.dot(x, wa_ref[...], preferred_element_type=jnp.float32) + ba_ref[...], 0.0)
    hb = jnp.maximum(
        jnp.dot(x, wb_ref[...], preferred_element_type=jnp.float32) + bb_ref[...], 0.0)
    o = (jnp.dot(ha, wa2_ref[...], preferred_element_type=jnp.float32)
         + jnp.dot(hb, wb2_ref[...], preferred_element_type=jnp.float32)
         + b2_ref[...])
    o_ref[...] = jnp.maximum(o, 0.0)


def _gin_mlp(h, p_ref, w1_ref, b1_ref, s1_ref, t1_ref, w2_ref, b2_ref,
             s2_ref, t2_ref):
    z = h + p_ref[0] + p_ref[1]
    y = jnp.dot(z, w1_ref[...], preferred_element_type=jnp.float32) + b1_ref[...]
    y = jnp.maximum(y * (s1_ref[...] * _BN_INV) + t1_ref[...], 0.0)
    m = jnp.dot(y, w2_ref[...], preferred_element_type=jnp.float32) + b2_ref[...]
    return m * (s2_ref[...] * _BN_INV) + t2_ref[...]


def _gin0_body(h_ref, p_ref, w1_ref, b1_ref, s1_ref, t1_ref, w2_ref, b2_ref,
               s2_ref, t2_ref, o_ref):
    h = h_ref[...]
    o_ref[...] = _gin_mlp(h, p_ref, w1_ref, b1_ref, s1_ref, t1_ref,
                          w2_ref, b2_ref, s2_ref, t2_ref) + h


def _gin1_pool_body(h_ref, p_ref, r_ref, batch_ref, w1_ref, b1_ref, s1_ref,
                    t1_ref, w2_ref, b2_ref, s2_ref, t2_ref, wp1_ref, bp1_ref,
                    wp2_ref, bp2_ref, o_ref, acc_ref, cnt_ref):
    i = pl.program_id(0)
    h = h_ref[...]
    g = _gin_mlp(h, p_ref, w1_ref, b1_ref, s1_ref, t1_ref,
                 w2_ref, b2_ref, s2_ref, t2_ref) + h + r_ref[...]

    b = batch_ref[0]
    gid = lax.broadcasted_iota(jnp.int32, (NG, BN), 0)
    oh = (gid == b).astype(jnp.float32)
    part = jnp.dot(oh, g, preferred_element_type=jnp.float32)
    csum = jnp.sum(oh, axis=1, keepdims=True)

    @pl.when(i == 0)
    def _():
        acc_ref[...] = part
        cnt_ref[...] = csum

    @pl.when(i > 0)
    def _():
        acc_ref[...] += part
        cnt_ref[...] += csum

    @pl.when(i == GRID - 1)
    def _():
        pooled = acc_ref[...] / jnp.maximum(cnt_ref[...], 1.0)
        u = jnp.maximum(
            jnp.dot(pooled, wp1_ref[...], preferred_element_type=jnp.float32)
            + bp1_ref[...], 0.0)
        v = (jnp.dot(u, wp2_ref[...], preferred_element_type=jnp.float32)
             + bp2_ref[...])
        mx = jnp.max(v, axis=1, keepdims=True)
        lse = jnp.log(jnp.sum(jnp.exp(v - mx), axis=1, keepdims=True)) + mx
        o_ref[...] = v - lse


def _full(shape):
    return pl.BlockSpec(shape, lambda i: tuple(0 for _ in shape))


def _pre_mlp(x_pad, wa, ba, wb, bb, wa2, wb2, b2):
    return pl.pallas_call(
        _pre_body,
        grid=(GRID,),
        in_specs=[
            pl.BlockSpec((BN, D), lambda i: (i, 0)),
            _full((D, 16)), _full((1, 16)),
            _full((D, 16)), _full((1, 16)),
            _full((16, H)), _full((16, H)), _full((1, H)),
        ],
        out_specs=pl.BlockSpec((BN, H), lambda i: (i, 0)),
        out_shape=jax.ShapeDtypeStruct((N_PAD, H), jnp.float32),
    )(x_pad, wa, ba, wb, bb, wa2, wb2, b2)


def _gin0(h, p, w1, b1, s1, t1, w2, b2, s2, t2):
    return pl.pallas_call(
        _gin0_body,
        grid=(GRID,),
        in_specs=[
            pl.BlockSpec((BN, H), lambda i: (i, 0)),
            pl.BlockSpec((2, BN, H), lambda i: (0, i, 0)),
            _full((H, H)), _full((1, H)), _full((1, H)), _full((1, H)),
            _full((H, H)), _full((1, H)), _full((1, H)), _full((1, H)),
        ],
        out_specs=pl.BlockSpec((BN, H), lambda i: (i, 0)),
        out_shape=jax.ShapeDtypeStruct((N_PAD, H), jnp.float32),
    )(h, p, w1, b1, s1, t1, w2, b2, s2, t2)


def _gin1_pool(h, p, r, batch3, w1, b1, s1, t1, w2, b2, s2, t2,
               wp1, bp1, wp2, bp2):
    return pl.pallas_call(
        _gin1_pool_body,
        grid=(GRID,),
        in_specs=[
            pl.BlockSpec((BN, H), lambda i: (i, 0)),
            pl.BlockSpec((2, BN, H), lambda i: (0, i, 0)),
            pl.BlockSpec((BN, H), lambda i: (i, 0)),
            pl.BlockSpec((1, 1, BN), lambda i: (i, 0, 0)),
            _full((H, H)), _full((1, H)), _full((1, H)), _full((1, H)),
            _full((H, H)), _full((1, H)), _full((1, H)), _full((1, H)),
            _full((H, 16)), _full((1, 16)), _full((16, 7)), _full((1, 7)),
        ],
        out_specs=_full((NG, 7)),
        out_shape=jax.ShapeDtypeStruct((NG, 7), jnp.float32),
        scratch_shapes=[
            pltpu.VMEM((NG, H), jnp.float32),
            pltpu.VMEM((NG, 1), jnp.float32),
        ],
    )(h, p, r, batch3, w1, b1, s1, t1, w2, b2, s2, t2, wp1, bp1, wp2, bp2)


def kernel(x, edge_index, batch, w_pre1, b_pre1, w_pre2, b_pre2, w_pre3,
           b_pre3, w_pre4, b_pre4, gin0_w1, gin0_b1, gin0_bng, gin0_bnb,
           gin0_w2, gin0_b2, gin1_w1, gin1_b1, gin1_bng, gin1_bnb, gin1_w2,
           gin1_b2, bn0_g, bn0_b, bn1_g, bn1_b, w_post1, b_post1, w_post2,
           b_post2):
    f32 = jnp.float32
    x_pad = jnp.pad(x, ((0, N_PAD - N), (0, 0)))
    src3 = jnp.concatenate(
        [edge_index[0], jnp.zeros((E_PAD - E,), jnp.int32)]
    ).reshape(NTILES, NCH, CHUNK)
    dst3 = jnp.concatenate(
        [edge_index[1], jnp.full((E_PAD - E,), N_PAD - 1, jnp.int32)]
    ).reshape(NTILES, NCH, CHUNK)
    batch3 = jnp.concatenate(
        [batch, jnp.full((N_PAD - N,), NG, jnp.int32)]
    ).reshape(GRID, 1, BN)

    wa = jnp.zeros((D, 16), f32).at[: D - CFEA].set(w_pre3.T)
    wb = jnp.zeros((D, 16), f32).at[D - CFEA:].set(w_pre1.T)
    wa2 = jnp.zeros((16, H), f32).at[:, : H // 2].set(w_pre4.T)
    wb2 = jnp.zeros((16, H), f32).at[:, H // 2:].set(w_pre2.T)
    b2c = jnp.concatenate([b_pre4, b_pre2]).reshape(1, H)

    def row(v):
        return v.reshape(1, -1)

    new_x = _pre_mlp(x_pad, wa, row(b_pre3), wb, row(b_pre1), wa2, wb2, b2c)

    p0 = _sc_agg(new_x, src3, dst3)
    g0 = _gin0(new_x, p0, gin0_w1.T, row(gin0_b1), row(gin0_bng),
               row(gin0_bnb), gin0_w2.T, row(gin0_b2), row(bn0_g), row(bn0_b))

    p1 = _sc_agg(g0, src3, dst3)
    return _gin1_pool(g0, p1, new_x, batch3, gin1_w1.T, row(gin1_b1),
                      row(gin1_bng), row(gin1_bnb), gin1_w2.T, row(gin1_b2),
                      row(bn1_g), row(bn1_b), w_post1.T, row(b_post1),
                      w_post2.T, row(b_post2))


# speedup vs baseline: 5.4054x; 1.0005x over previous
import functools

import jax
import jax.numpy as jnp
from jax import lax
from jax.experimental import pallas as pl
from jax.experimental.pallas import tpu as pltpu
from jax.experimental.pallas import tpu_sc as plsc

N = 10000
E = 320000
D = 128
CFEA = 2
H = 64
NG = 64

BN = 1024
N_PAD = 10240
GRID = N_PAD // BN

NTILES = 32
EPT = 10240
E_PAD = NTILES * EPT
CHUNK = 128
NCH = EPT // CHUNK
NPAIR = NCH // 2
RPS = N_PAD // 16

_BN_INV = (1.0 + 1e-5) ** -0.5


@functools.cache
def _make_sc_agg():
    mesh = plsc.VectorSubcoreMesh(
        core_axis_name="c", subcore_axis_name="s", num_cores=2, num_subcores=16
    )
    return pl.kernel(
        _sc_agg_body,
        out_type=jax.ShapeDtypeStruct((2, N_PAD, H), jnp.float32),
        mesh=mesh,
        scratch_types=[
            pltpu.VMEM((NCH, CHUNK), jnp.int32),
            pltpu.VMEM((NCH, CHUNK), jnp.int32),
            pltpu.VMEM((2, CHUNK, H), jnp.float32),
            pltpu.VMEM_SHARED((N_PAD, H), jnp.float32),
            pltpu.SemaphoreType.DMA,
            pltpu.SemaphoreType.DMA,
        ],
        compiler_params=pltpu.CompilerParams(use_tc_tiling_on_sc=False),
    )


def _sc_agg(h, src3, dst3):
    return _make_sc_agg()(h, src3, dst3)


def _sc_agg_body(h_hbm, src_hbm, dst_hbm, out_hbm, src_v, dst_v, buf, acc, sem0, sem1):
    c = lax.axis_index("c")
    s = lax.axis_index("s")
    w = c * 16 + s

    pltpu.sync_copy(src_hbm.at[w], src_v)
    pltpu.sync_copy(dst_hbm.at[w], dst_v)

    def _zrow(i, carry):
        for k4 in range(4):
            buf[0, i, pl.ds(k4 * 16, 16)] = jnp.zeros((16,), jnp.float32)
        return carry

    lax.fori_loop(0, CHUNK, _zrow, 0)

    def _crow(j, carry):
        pltpu.sync_copy(buf.at[0],
                        acc.at[pl.ds(s * RPS + j * CHUNK, CHUNK)])
        return carry

    lax.fori_loop(0, RPS // CHUNK, _crow, 0)
    plsc.subcore_barrier()

    pltpu.async_copy(h_hbm.at[src_v.at[0]], buf.at[0], sem0)

    def _pair(jj, carry):
        j0 = 2 * jj
        cp1 = pltpu.async_copy(h_hbm.at[src_v.at[j0 + 1]], buf.at[1], sem1)
        pltpu.make_async_copy(
            h_hbm.at[src_v.at[j0]], buf.at[0], sem0).wait()
        pltpu.sync_copy(buf.at[0], acc.at[dst_v.at[j0]], add=True)

        @pl.when(jj < NPAIR - 1)
        def _():
            pltpu.async_copy(h_hbm.at[src_v.at[j0 + 2]], buf.at[0], sem0)

        cp1.wait()
        pltpu.sync_copy(buf.at[1], acc.at[dst_v.at[j0 + 1]], add=True)
        return carry

    lax.fori_loop(0, NPAIR, _pair, 0)
    plsc.subcore_barrier()

    r0 = s * RPS
    pltpu.sync_copy(acc.at[pl.ds(r0, RPS)], out_hbm.at[c, pl.ds(r0, RPS)])


def _pre_body(x_ref, wa_ref, ba_ref, wb_ref, bb_ref, wa2_ref, wb2_ref, b2_ref,
              o_ref):
    x = x_ref[...]
    ha = jnp.maximum(
        jnp.dot(x, wa_ref[...], preferred_element_type=jnp.float32) + ba_ref[...], 0.0)
    hb = jnp.maximum(
        jnp.dot(x, wb_ref[...], preferred_element_type=jnp.float32) + bb_ref[...], 0.0)
    o = (jnp.dot(ha, wa2_ref[...], preferred_element_type=jnp.float32)
         + jnp.dot(hb, wb2_ref[...], preferred_element_type=jnp.float32)
         + b2_ref[...])
    o_ref[...] = jnp.maximum(o, 0.0)


def _gin_mlp(h, p_ref, w1_ref, b1_ref, s1_ref, t1_ref, w2_ref, b2_ref,
             s2_ref, t2_ref):
    z = h + p_ref[0] + p_ref[1]
    y = jnp.dot(z, w1_ref[...], preferred_element_type=jnp.float32) + b1_ref[...]
    y = jnp.maximum(y * (s1_ref[...] * _BN_INV) + t1_ref[...], 0.0)
    m = jnp.dot(y, w2_ref[...], preferred_element_type=jnp.float32) + b2_ref[...]
    return m * (s2_ref[...] * _BN_INV) + t2_ref[...]


def _gin0_body(h_ref, p_ref, w1_ref, b1_ref, s1_ref, t1_ref, w2_ref, b2_ref,
               s2_ref, t2_ref, o_ref):
    h = h_ref[...]
    o_ref[...] = _gin_mlp(h, p_ref, w1_ref, b1_ref, s1_ref, t1_ref,
                          w2_ref, b2_ref, s2_ref, t2_ref) + h


def _gin1_pool_body(h_ref, p_ref, r_ref, batch_ref, w1_ref, b1_ref, s1_ref,
                    t1_ref, w2_ref, b2_ref, s2_ref, t2_ref, wp1_ref, bp1_ref,
                    wp2_ref, bp2_ref, o_ref, acc_ref, cnt_ref):
    i = pl.program_id(0)
    h = h_ref[...]
    g = _gin_mlp(h, p_ref, w1_ref, b1_ref, s1_ref, t1_ref,
                 w2_ref, b2_ref, s2_ref, t2_ref) + h + r_ref[...]

    b = batch_ref[0]
    gid = lax.broadcasted_iota(jnp.int32, (NG, BN), 0)
    oh = (gid == b).astype(jnp.float32)
    part = jnp.dot(oh, g, preferred_element_type=jnp.float32)
    csum = jnp.sum(oh, axis=1, keepdims=True)

    @pl.when(i == 0)
    def _():
        acc_ref[...] = part
        cnt_ref[...] = csum

    @pl.when(i > 0)
    def _():
        acc_ref[...] += part
        cnt_ref[...] += csum

    @pl.when(i == GRID - 1)
    def _():
        pooled = acc_ref[...] / jnp.maximum(cnt_ref[...], 1.0)
        u = jnp.maximum(
            jnp.dot(pooled, wp1_ref[...], preferred_element_type=jnp.float32)
            + bp1_ref[...], 0.0)
        v = (jnp.dot(u, wp2_ref[...], preferred_element_type=jnp.float32)
             + bp2_ref[...])
        mx = jnp.max(v, axis=1, keepdims=True)
        lse = jnp.log(jnp.sum(jnp.exp(v - mx), axis=1, keepdims=True)) + mx
        o_ref[...] = v - lse


def _full(shape):
    return pl.BlockSpec(shape, lambda i: tuple(0 for _ in shape))


def _pre_mlp(x_pad, wa, ba, wb, bb, wa2, wb2, b2):
    return pl.pallas_call(
        _pre_body,
        grid=(GRID,),
        in_specs=[
            pl.BlockSpec((BN, D), lambda i: (i, 0)),
            _full((D, 16)), _full((1, 16)),
            _full((D, 16)), _full((1, 16)),
            _full((16, H)), _full((16, H)), _full((1, H)),
        ],
        out_specs=pl.BlockSpec((BN, H), lambda i: (i, 0)),
        out_shape=jax.ShapeDtypeStruct((N_PAD, H), jnp.float32),
    )(x_pad, wa, ba, wb, bb, wa2, wb2, b2)


def _gin0(h, p, w1, b1, s1, t1, w2, b2, s2, t2):
    return pl.pallas_call(
        _gin0_body,
        grid=(GRID,),
        in_specs=[
            pl.BlockSpec((BN, H), lambda i: (i, 0)),
            pl.BlockSpec((2, BN, H), lambda i: (0, i, 0)),
            _full((H, H)), _full((1, H)), _full((1, H)), _full((1, H)),
            _full((H, H)), _full((1, H)), _full((1, H)), _full((1, H)),
        ],
        out_specs=pl.BlockSpec((BN, H), lambda i: (i, 0)),
        out_shape=jax.ShapeDtypeStruct((N_PAD, H), jnp.float32),
    )(h, p, w1, b1, s1, t1, w2, b2, s2, t2)


def _gin1_pool(h, p, r, batch3, w1, b1, s1, t1, w2, b2, s2, t2,
               wp1, bp1, wp2, bp2):
    return pl.pallas_call(
        _gin1_pool_body,
        grid=(GRID,),
        in_specs=[
            pl.BlockSpec((BN, H), lambda i: (i, 0)),
            pl.BlockSpec((2, BN, H), lambda i: (0, i, 0)),
            pl.BlockSpec((BN, H), lambda i: (i, 0)),
            pl.BlockSpec((1, 1, BN), lambda i: (i, 0, 0)),
            _full((H, H)), _full((1, H)), _full((1, H)), _full((1, H)),
            _full((H, H)), _full((1, H)), _full((1, H)), _full((1, H)),
            _full((H, 16)), _full((1, 16)), _full((16, 7)), _full((1, 7)),
        ],
        out_specs=_full((NG, 7)),
        out_shape=jax.ShapeDtypeStruct((NG, 7), jnp.float32),
        scratch_shapes=[
            pltpu.VMEM((NG, H), jnp.float32),
            pltpu.VMEM((NG, 1), jnp.float32),
        ],
    )(h, p, r, batch3, w1, b1, s1, t1, w2, b2, s2, t2, wp1, bp1, wp2, bp2)


def kernel(x, edge_index, batch, w_pre1, b_pre1, w_pre2, b_pre2, w_pre3,
           b_pre3, w_pre4, b_pre4, gin0_w1, gin0_b1, gin0_bng, gin0_bnb,
           gin0_w2, gin0_b2, gin1_w1, gin1_b1, gin1_bng, gin1_bnb, gin1_w2,
           gin1_b2, bn0_g, bn0_b, bn1_g, bn1_b, w_post1, b_post1, w_post2,
           b_post2):
    f32 = jnp.float32
    x_pad = jnp.pad(x, ((0, N_PAD - N), (0, 0)))
    src3 = jnp.concatenate(
        [edge_index[0], jnp.zeros((E_PAD - E,), jnp.int32)]
    ).reshape(NTILES, NCH, CHUNK)
    dst3 = jnp.concatenate(
        [edge_index[1], jnp.full((E_PAD - E,), N_PAD - 1, jnp.int32)]
    ).reshape(NTILES, NCH, CHUNK)
    batch3 = jnp.concatenate(
        [batch, jnp.full((N_PAD - N,), NG, jnp.int32)]
    ).reshape(GRID, 1, BN)

    wa = jnp.zeros((D, 16), f32).at[: D - CFEA].set(w_pre3.T)
    wb = jnp.zeros((D, 16), f32).at[D - CFEA:].set(w_pre1.T)
    wa2 = jnp.zeros((16, H), f32).at[:, : H // 2].set(w_pre4.T)
    wb2 = jnp.zeros((16, H), f32).at[:, H // 2:].set(w_pre2.T)
    b2c = jnp.concatenate([b_pre4, b_pre2]).reshape(1, H)

    def row(v):
        return v.reshape(1, -1)

    new_x = _pre_mlp(x_pad, wa, row(b_pre3), wb, row(b_pre1), wa2, wb2, b2c)

    p0 = _sc_agg(new_x, src3, dst3)
    g0 = _gin0(new_x, p0, gin0_w1.T, row(gin0_b1), row(gin0_bng),
               row(gin0_bnb), gin0_w2.T, row(gin0_b2), row(bn0_g), row(bn0_b))

    p1 = _sc_agg(g0, src3, dst3)
    return _gin1_pool(g0, p1, new_x, batch3, gin1_w1.T, row(gin1_b1),
                      row(gin1_bng), row(gin1_bnb), gin1_w2.T, row(gin1_b2),
                      row(bn1_g), row(bn1_b), w_post1.T, row(b_post1),
                      w_post2.T, row(b_post2))


# speedup vs baseline: 11.3341x; 2.0968x over previous
import functools

import jax
import jax.numpy as jnp
from jax import lax
from jax.experimental import pallas as pl
from jax.experimental.pallas import tpu as pltpu
from jax.experimental.pallas import tpu_sc as plsc

N = 10000
E = 320000
D = 128
CFEA = 2
H = 64
NG = 64

BN = 1024
N_PAD = 10240
GRID = N_PAD // BN

NTILES = 32
EPT = 10240
E_PAD = NTILES * EPT
CHUNK = 128
NCH = EPT // CHUNK
NPAIR = NCH // 2
RPS = N_PAD // 16

_BN_INV = (1.0 + 1e-5) ** -0.5


@functools.cache
def _make_sc_agg():
    mesh = plsc.VectorSubcoreMesh(
        core_axis_name="c", subcore_axis_name="s", num_cores=2, num_subcores=16
    )
    return pl.kernel(
        _sc_agg_body,
        out_type=jax.ShapeDtypeStruct((2, N_PAD, H), jnp.float32),
        mesh=mesh,
        scratch_types=[
            pltpu.VMEM((NCH, CHUNK), jnp.int32),
            pltpu.VMEM((NCH, CHUNK), jnp.int32),
            pltpu.VMEM((2, CHUNK, H), jnp.float32),
            pltpu.VMEM_SHARED((N_PAD, H), jnp.float32),
            pltpu.VMEM_SHARED((N_PAD, H), jnp.float32),
            pltpu.SemaphoreType.DMA,
            pltpu.SemaphoreType.DMA,
        ],
        compiler_params=pltpu.CompilerParams(use_tc_tiling_on_sc=False),
    )


def _sc_agg(h, src3, dst3):
    return _make_sc_agg()(h, src3, dst3)


def _sc_agg_body(h_hbm, src_hbm, dst_hbm, out_hbm, src_v, dst_v, buf, acc,
                 h_sp, sem0, sem1):
    c = lax.axis_index("c")
    s = lax.axis_index("s")
    w = c * 16 + s

    r0s = s * RPS
    pltpu.sync_copy(h_hbm.at[pl.ds(r0s, RPS)], h_sp.at[pl.ds(r0s, RPS)])

    pltpu.sync_copy(src_hbm.at[w], src_v)
    pltpu.sync_copy(dst_hbm.at[w], dst_v)

    def _zrow(i, carry):
        for k4 in range(4):
            buf[0, i, pl.ds(k4 * 16, 16)] = jnp.zeros((16,), jnp.float32)
        return carry

    lax.fori_loop(0, CHUNK, _zrow, 0)

    def _crow(j, carry):
        pltpu.sync_copy(buf.at[0],
                        acc.at[pl.ds(s * RPS + j * CHUNK, CHUNK)])
        return carry

    lax.fori_loop(0, RPS // CHUNK, _crow, 0)
    plsc.subcore_barrier()

    pltpu.async_copy(h_sp.at[src_v.at[0]], buf.at[0], sem0)

    def _pair(jj, carry):
        j0 = 2 * jj
        cp1 = pltpu.async_copy(h_sp.at[src_v.at[j0 + 1]], buf.at[1], sem1)
        pltpu.make_async_copy(
            h_sp.at[src_v.at[j0]], buf.at[0], sem0).wait()
        pltpu.sync_copy(buf.at[0], acc.at[dst_v.at[j0]], add=True)

        @pl.when(jj < NPAIR - 1)
        def _():
            pltpu.async_copy(h_sp.at[src_v.at[j0 + 2]], buf.at[0], sem0)

        cp1.wait()
        pltpu.sync_copy(buf.at[1], acc.at[dst_v.at[j0 + 1]], add=True)
        return carry

    lax.fori_loop(0, NPAIR, _pair, 0)
    plsc.subcore_barrier()

    r0 = s * RPS
    pltpu.sync_copy(acc.at[pl.ds(r0, RPS)], out_hbm.at[c, pl.ds(r0, RPS)])


def _pre_body(x_ref, wa_ref, ba_ref, wb_ref, bb_ref, wa2_ref, wb2_ref, b2_ref,
              o_ref):
    x = x_ref[...]
    ha = jnp.maximum(
        jnp.dot(x, wa_ref[...], preferred_element_type=jnp.float32) + ba_ref[...], 0.0)
    hb = jnp.maximum(
        jnp.dot(x, wb_ref[...], preferred_element_type=jnp.float32) + bb_ref[...], 0.0)
    o = (jnp.dot(ha, wa2_ref[...], preferred_element_type=jnp.float32)
         + jnp.dot(hb, wb2_ref[...], preferred_element_type=jnp.float32)
         + b2_ref[...])
    o_ref[...] = jnp.maximum(o, 0.0)


def _gin_mlp(h, p_ref, w1_ref, b1_ref, s1_ref, t1_ref, w2_ref, b2_ref,
             s2_ref, t2_ref):
    z = h + p_ref[0] + p_ref[1]
    y = jnp.dot(z, w1_ref[...], preferred_element_type=jnp.float32) + b1_ref[...]
    y = jnp.maximum(y * (s1_ref[...] * _BN_INV) + t1_ref[...], 0.0)
    m = jnp.dot(y, w2_ref[...], preferred_element_type=jnp.float32) + b2_ref[...]
    return m * (s2_ref[...] * _BN_INV) + t2_ref[...]


def _gin0_body(h_ref, p_ref, w1_ref, b1_ref, s1_ref, t1_ref, w2_ref, b2_ref,
               s2_ref, t2_ref, o_ref):
    h = h_ref[...]
    o_ref[...] = _gin_mlp(h, p_ref, w1_ref, b1_ref, s1_ref, t1_ref,
                          w2_ref, b2_ref, s2_ref, t2_ref) + h


def _gin1_pool_body(h_ref, p_ref, r_ref, batch_ref, w1_ref, b1_ref, s1_ref,
                    t1_ref, w2_ref, b2_ref, s2_ref, t2_ref, wp1_ref, bp1_ref,
                    wp2_ref, bp2_ref, o_ref, acc_ref, cnt_ref):
    i = pl.program_id(0)
    h = h_ref[...]
    g = _gin_mlp(h, p_ref, w1_ref, b1_ref, s1_ref, t1_ref,
                 w2_ref, b2_ref, s2_ref, t2_ref) + h + r_ref[...]

    b = batch_ref[0]
    gid = lax.broadcasted_iota(jnp.int32, (NG, BN), 0)
    oh = (gid == b).astype(jnp.float32)
    part = jnp.dot(oh, g, preferred_element_type=jnp.float32)
    csum = jnp.sum(oh, axis=1, keepdims=True)

    @pl.when(i == 0)
    def _():
        acc_ref[...] = part
        cnt_ref[...] = csum

    @pl.when(i > 0)
    def _():
        acc_ref[...] += part
        cnt_ref[...] += csum

    @pl.when(i == GRID - 1)
    def _():
        pooled = acc_ref[...] / jnp.maximum(cnt_ref[...], 1.0)
        u = jnp.maximum(
            jnp.dot(pooled, wp1_ref[...], preferred_element_type=jnp.float32)
            + bp1_ref[...], 0.0)
        v = (jnp.dot(u, wp2_ref[...], preferred_element_type=jnp.float32)
             + bp2_ref[...])
        mx = jnp.max(v, axis=1, keepdims=True)
        lse = jnp.log(jnp.sum(jnp.exp(v - mx), axis=1, keepdims=True)) + mx
        o_ref[...] = v - lse


def _full(shape):
    return pl.BlockSpec(shape, lambda i: tuple(0 for _ in shape))


def _pre_mlp(x_pad, wa, ba, wb, bb, wa2, wb2, b2):
    return pl.pallas_call(
        _pre_body,
        grid=(GRID,),
        in_specs=[
            pl.BlockSpec((BN, D), lambda i: (i, 0)),
            _full((D, 16)), _full((1, 16)),
            _full((D, 16)), _full((1, 16)),
            _full((16, H)), _full((16, H)), _full((1, H)),
        ],
        out_specs=pl.BlockSpec((BN, H), lambda i: (i, 0)),
        out_shape=jax.ShapeDtypeStruct((N_PAD, H), jnp.float32),
    )(x_pad, wa, ba, wb, bb, wa2, wb2, b2)


def _gin0(h, p, w1, b1, s1, t1, w2, b2, s2, t2):
    return pl.pallas_call(
        _gin0_body,
        grid=(GRID,),
        in_specs=[
            pl.BlockSpec((BN, H), lambda i: (i, 0)),
            pl.BlockSpec((2, BN, H), lambda i: (0, i, 0)),
            _full((H, H)), _full((1, H)), _full((1, H)), _full((1, H)),
            _full((H, H)), _full((1, H)), _full((1, H)), _full((1, H)),
        ],
        out_specs=pl.BlockSpec((BN, H), lambda i: (i, 0)),
        out_shape=jax.ShapeDtypeStruct((N_PAD, H), jnp.float32),
    )(h, p, w1, b1, s1, t1, w2, b2, s2, t2)


def _gin1_pool(h, p, r, batch3, w1, b1, s1, t1, w2, b2, s2, t2,
               wp1, bp1, wp2, bp2):
    return pl.pallas_call(
        _gin1_pool_body,
        grid=(GRID,),
        in_specs=[
            pl.BlockSpec((BN, H), lambda i: (i, 0)),
            pl.BlockSpec((2, BN, H), lambda i: (0, i, 0)),
            pl.BlockSpec((BN, H), lambda i: (i, 0)),
            pl.BlockSpec((1, 1, BN), lambda i: (i, 0, 0)),
            _full((H, H)), _full((1, H)), _full((1, H)), _full((1, H)),
            _full((H, H)), _full((1, H)), _full((1, H)), _full((1, H)),
            _full((H, 16)), _full((1, 16)), _full((16, 7)), _full((1, 7)),
        ],
        out_specs=_full((NG, 7)),
        out_shape=jax.ShapeDtypeStruct((NG, 7), jnp.float32),
        scratch_shapes=[
            pltpu.VMEM((NG, H), jnp.float32),
            pltpu.VMEM((NG, 1), jnp.float32),
        ],
    )(h, p, r, batch3, w1, b1, s1, t1, w2, b2, s2, t2, wp1, bp1, wp2, bp2)


def kernel(x, edge_index, batch, w_pre1, b_pre1, w_pre2, b_pre2, w_pre3,
           b_pre3, w_pre4, b_pre4, gin0_w1, gin0_b1, gin0_bng, gin0_bnb,
           gin0_w2, gin0_b2, gin1_w1, gin1_b1, gin1_bng, gin1_bnb, gin1_w2,
           gin1_b2, bn0_g, bn0_b, bn1_g, bn1_b, w_post1, b_post1, w_post2,
           b_post2):
    f32 = jnp.float32
    x_pad = jnp.pad(x, ((0, N_PAD - N), (0, 0)))
    src3 = jnp.concatenate(
        [edge_index[0], jnp.zeros((E_PAD - E,), jnp.int32)]
    ).reshape(NTILES, NCH, CHUNK)
    dst3 = jnp.concatenate(
        [edge_index[1], jnp.full((E_PAD - E,), N_PAD - 1, jnp.int32)]
    ).reshape(NTILES, NCH, CHUNK)
    batch3 = jnp.concatenate(
        [batch, jnp.full((N_PAD - N,), NG, jnp.int32)]
    ).reshape(GRID, 1, BN)

    wa = jnp.zeros((D, 16), f32).at[: D - CFEA].set(w_pre3.T)
    wb = jnp.zeros((D, 16), f32).at[D - CFEA:].set(w_pre1.T)
    wa2 = jnp.zeros((16, H), f32).at[:, : H // 2].set(w_pre4.T)
    wb2 = jnp.zeros((16, H), f32).at[:, H // 2:].set(w_pre2.T)
    b2c = jnp.concatenate([b_pre4, b_pre2]).reshape(1, H)

    def row(v):
        return v.reshape(1, -1)

    new_x = _pre_mlp(x_pad, wa, row(b_pre3), wb, row(b_pre1), wa2, wb2, b2c)

    p0 = _sc_agg(new_x, src3, dst3)
    g0 = _gin0(new_x, p0, gin0_w1.T, row(gin0_b1), row(gin0_bng),
               row(gin0_bnb), gin0_w2.T, row(gin0_b2), row(bn0_g), row(bn0_b))

    p1 = _sc_agg(g0, src3, dst3)
    return _gin1_pool(g0, p1, new_x, batch3, gin1_w1.T, row(gin1_b1),
                      row(gin1_bng), row(gin1_bnb), gin1_w2.T, row(gin1_b2),
                      row(bn1_g), row(bn1_b), w_post1.T, row(b_post1),
                      w_post2.T, row(b_post2))


# speedup vs baseline: 12.7002x; 1.1205x over previous
import functools

import jax
import jax.numpy as jnp
from jax import lax
from jax.experimental import pallas as pl
from jax.experimental.pallas import tpu as pltpu
from jax.experimental.pallas import tpu_sc as plsc

N = 10000
E = 320000
D = 128
CFEA = 2
H = 64
NG = 64

BN = 1024
N_PAD = 10240
GRID = N_PAD // BN

NTILES = 32
EPT = 10240
E_PAD = NTILES * EPT
CHUNK = 128
NCH = EPT // CHUNK
NBUF = 3
RPS = N_PAD // 16

_BN_INV = (1.0 + 1e-5) ** -0.5


@functools.cache
def _make_sc_agg():
    mesh = plsc.VectorSubcoreMesh(
        core_axis_name="c", subcore_axis_name="s", num_cores=2, num_subcores=16
    )
    return pl.kernel(
        _sc_agg_body,
        out_type=jax.ShapeDtypeStruct((2, N_PAD, H), jnp.float32),
        mesh=mesh,
        scratch_types=[
            pltpu.VMEM((NCH, CHUNK), jnp.int32),
            pltpu.VMEM((NCH, CHUNK), jnp.int32),
            pltpu.VMEM((NBUF, CHUNK, H), jnp.float32),
            pltpu.VMEM_SHARED((N_PAD, H), jnp.float32),
            pltpu.VMEM_SHARED((N_PAD, H), jnp.float32),
        ] + [pltpu.SemaphoreType.DMA] * (2 * NBUF),
        compiler_params=pltpu.CompilerParams(use_tc_tiling_on_sc=False),
    )


def _sc_agg(h, src3, dst3):
    return _make_sc_agg()(h, src3, dst3)


def _sc_agg_body(h_hbm, src_hbm, dst_hbm, out_hbm, src_v, dst_v, buf, acc,
                 h_sp, g0, g1, g2, s0, s1, s2):
    gsem = (g0, g1, g2)
    ssem = (s0, s1, s2)
    c = lax.axis_index("c")
    s = lax.axis_index("s")
    w = c * 16 + s
    r0s = s * RPS

    cp_h = pltpu.async_copy(h_hbm.at[pl.ds(r0s, RPS)],
                            h_sp.at[pl.ds(r0s, RPS)], g0)
    cp_s = pltpu.async_copy(src_hbm.at[w], src_v, g1)
    cp_d = pltpu.async_copy(dst_hbm.at[w], dst_v, g2)

    def _zrow(i, carry):
        for k4 in range(4):
            buf[0, i, pl.ds(k4 * 16, 16)] = jnp.zeros((16,), jnp.float32)
        return carry

    lax.fori_loop(0, CHUNK, _zrow, 0)

    def _crow(j, carry):
        pltpu.sync_copy(buf.at[0],
                        acc.at[pl.ds(r0s + j * CHUNK, CHUNK)])
        return carry

    lax.fori_loop(0, RPS // CHUNK, _crow, 0)
    cp_h.wait()
    cp_s.wait()
    cp_d.wait()
    plsc.subcore_barrier()

    pltpu.async_copy(h_sp.at[src_v.at[0]], buf.at[0], gsem[0])
    pltpu.async_copy(h_sp.at[src_v.at[1]], buf.at[1], gsem[1])

    def _group(gg, carry):
        for b in range(NBUF):
            j = NBUF * gg + b
            bn = (b + 2) % NBUF
            pltpu.make_async_copy(
                h_sp.at[src_v.at[j]], buf.at[b], gsem[b]).wait()
            pltpu.async_copy(buf.at[b], acc.at[dst_v.at[j]], ssem[b],
                             add=True)

            @pl.when(j >= 1)
            def _(j=j, bn=bn):
                pltpu.make_async_copy(
                    buf.at[bn], acc.at[dst_v.at[j]], ssem[bn]).wait()

            pltpu.async_copy(h_sp.at[src_v.at[j + 2]], buf.at[bn], gsem[bn])
        return carry

    lax.fori_loop(0, (NCH - 2) // NBUF, _group, 0)
    pltpu.make_async_copy(
        h_sp.at[src_v.at[NCH - 2]], buf.at[0], gsem[0]).wait()
    pltpu.async_copy(buf.at[0], acc.at[dst_v.at[NCH - 2]], ssem[0], add=True)
    pltpu.make_async_copy(
        h_sp.at[src_v.at[NCH - 1]], buf.at[1], gsem[1]).wait()
    pltpu.async_copy(buf.at[1], acc.at[dst_v.at[NCH - 1]], ssem[1], add=True)
    for b in range(NBUF):
        pltpu.make_async_copy(buf.at[b], acc.at[dst_v.at[0]], ssem[b]).wait()
    plsc.subcore_barrier()

    pltpu.sync_copy(acc.at[pl.ds(r0s, RPS)], out_hbm.at[c, pl.ds(r0s, RPS)])


def _pre_body(x_ref, wa_ref, ba_ref, wb_ref, bb_ref, wa2_ref, wb2_ref, b2_ref,
              o_ref):
    x = x_ref[...]
    ha = jnp.maximum(
        jnp.dot(x, wa_ref[...], preferred_element_type=jnp.float32) + ba_ref[...], 0.0)
    hb = jnp.maximum(
        jnp.dot(x, wb_ref[...], preferred_element_type=jnp.float32) + bb_ref[...], 0.0)
    o = (jnp.dot(ha, wa2_ref[...], preferred_element_type=jnp.float32)
         + jnp.dot(hb, wb2_ref[...], preferred_element_type=jnp.float32)
         + b2_ref[...])
    o_ref[...] = jnp.maximum(o, 0.0)


def _gin_mlp(h, p_ref, w1_ref, b1_ref, s1_ref, t1_ref, w2_ref, b2_ref,
             s2_ref, t2_ref):
    z = h + p_ref[0] + p_ref[1]
    y = jnp.dot(z, w1_ref[...], preferred_element_type=jnp.float32) + b1_ref[...]
    y = jnp.maximum(y * (s1_ref[...] * _BN_INV) + t1_ref[...], 0.0)
    m = jnp.dot(y, w2_ref[...], preferred_element_type=jnp.float32) + b2_ref[...]
    return m * (s2_ref[...] * _BN_INV) + t2_ref[...]


def _gin0_body(h_ref, p_ref, w1_ref, b1_ref, s1_ref, t1_ref, w2_ref, b2_ref,
               s2_ref, t2_ref, o_ref):
    h = h_ref[...]
    o_ref[...] = _gin_mlp(h, p_ref, w1_ref, b1_ref, s1_ref, t1_ref,
                          w2_ref, b2_ref, s2_ref, t2_ref) + h


def _gin1_pool_body(h_ref, p_ref, r_ref, batch_ref, w1_ref, b1_ref, s1_ref,
                    t1_ref, w2_ref, b2_ref, s2_ref, t2_ref, wp1_ref, bp1_ref,
                    wp2_ref, bp2_ref, o_ref, acc_ref, cnt_ref):
    i = pl.program_id(0)
    h = h_ref[...]
    g = _gin_mlp(h, p_ref, w1_ref, b1_ref, s1_ref, t1_ref,
                 w2_ref, b2_ref, s2_ref, t2_ref) + h + r_ref[...]

    b = batch_ref[0]
    gid = lax.broadcasted_iota(jnp.int32, (NG, BN), 0)
    oh = (gid == b).astype(jnp.float32)
    part = jnp.dot(oh, g, preferred_element_type=jnp.float32)
    csum = jnp.sum(oh, axis=1, keepdims=True)

    @pl.when(i == 0)
    def _():
        acc_ref[...] = part
        cnt_ref[...] = csum

    @pl.when(i > 0)
    def _():
        acc_ref[...] += part
        cnt_ref[...] += csum

    @pl.when(i == GRID - 1)
    def _():
        pooled = acc_ref[...] / jnp.maximum(cnt_ref[...], 1.0)
        u = jnp.maximum(
            jnp.dot(pooled, wp1_ref[...], preferred_element_type=jnp.float32)
            + bp1_ref[...], 0.0)
        v = (jnp.dot(u, wp2_ref[...], preferred_element_type=jnp.float32)
             + bp2_ref[...])
        mx = jnp.max(v, axis=1, keepdims=True)
        lse = jnp.log(jnp.sum(jnp.exp(v - mx), axis=1, keepdims=True)) + mx
        o_ref[...] = v - lse


def _full(shape):
    return pl.BlockSpec(shape, lambda i: tuple(0 for _ in shape))


def _pre_mlp(x_pad, wa, ba, wb, bb, wa2, wb2, b2):
    return pl.pallas_call(
        _pre_body,
        grid=(GRID,),
        in_specs=[
            pl.BlockSpec((BN, D), lambda i: (i, 0)),
            _full((D, 16)), _full((1, 16)),
            _full((D, 16)), _full((1, 16)),
            _full((16, H)), _full((16, H)), _full((1, H)),
        ],
        out_specs=pl.BlockSpec((BN, H), lambda i: (i, 0)),
        out_shape=jax.ShapeDtypeStruct((N_PAD, H), jnp.float32),
    )(x_pad, wa, ba, wb, bb, wa2, wb2, b2)


def _gin0(h, p, w1, b1, s1, t1, w2, b2, s2, t2):
    return pl.pallas_call(
        _gin0_body,
        grid=(GRID,),
        in_specs=[
            pl.BlockSpec((BN, H), lambda i: (i, 0)),
            pl.BlockSpec((2, BN, H), lambda i: (0, i, 0)),
            _full((H, H)), _full((1, H)), _full((1, H)), _full((1, H)),
            _full((H, H)), _full((1, H)), _full((1, H)), _full((1, H)),
        ],
        out_specs=pl.BlockSpec((BN, H), lambda i: (i, 0)),
        out_shape=jax.ShapeDtypeStruct((N_PAD, H), jnp.float32),
    )(h, p, w1, b1, s1, t1, w2, b2, s2, t2)


def _gin1_pool(h, p, r, batch3, w1, b1, s1, t1, w2, b2, s2, t2,
               wp1, bp1, wp2, bp2):
    return pl.pallas_call(
        _gin1_pool_body,
        grid=(GRID,),
        in_specs=[
            pl.BlockSpec((BN, H), lambda i: (i, 0)),
            pl.BlockSpec((2, BN, H), lambda i: (0, i, 0)),
            pl.BlockSpec((BN, H), lambda i: (i, 0)),
            pl.BlockSpec((1, 1, BN), lambda i: (i, 0, 0)),
            _full((H, H)), _full((1, H)), _full((1, H)), _full((1, H)),
            _full((H, H)), _full((1, H)), _full((1, H)), _full((1, H)),
            _full((H, 16)), _full((1, 16)), _full((16, 7)), _full((1, 7)),
        ],
        out_specs=_full((NG, 7)),
        out_shape=jax.ShapeDtypeStruct((NG, 7), jnp.float32),
        scratch_shapes=[
            pltpu.VMEM((NG, H), jnp.float32),
            pltpu.VMEM((NG, 1), jnp.float32),
        ],
    )(h, p, r, batch3, w1, b1, s1, t1, w2, b2, s2, t2, wp1, bp1, wp2, bp2)


def kernel(x, edge_index, batch, w_pre1, b_pre1, w_pre2, b_pre2, w_pre3,
           b_pre3, w_pre4, b_pre4, gin0_w1, gin0_b1, gin0_bng, gin0_bnb,
           gin0_w2, gin0_b2, gin1_w1, gin1_b1, gin1_bng, gin1_bnb, gin1_w2,
           gin1_b2, bn0_g, bn0_b, bn1_g, bn1_b, w_post1, b_post1, w_post2,
           b_post2):
    f32 = jnp.float32
    x_pad = jnp.pad(x, ((0, N_PAD - N), (0, 0)))
    src3 = jnp.concatenate(
        [edge_index[0], jnp.zeros((E_PAD - E,), jnp.int32)]
    ).reshape(NTILES, NCH, CHUNK)
    dst3 = jnp.concatenate(
        [edge_index[1], jnp.full((E_PAD - E,), N_PAD - 1, jnp.int32)]
    ).reshape(NTILES, NCH, CHUNK)
    batch3 = jnp.concatenate(
        [batch, jnp.full((N_PAD - N,), NG, jnp.int32)]
    ).reshape(GRID, 1, BN)

    wa = jnp.zeros((D, 16), f32).at[: D - CFEA].set(w_pre3.T)
    wb = jnp.zeros((D, 16), f32).at[D - CFEA:].set(w_pre1.T)
    wa2 = jnp.zeros((16, H), f32).at[:, : H // 2].set(w_pre4.T)
    wb2 = jnp.zeros((16, H), f32).at[:, H // 2:].set(w_pre2.T)
    b2c = jnp.concatenate([b_pre4, b_pre2]).reshape(1, H)

    def row(v):
        return v.reshape(1, -1)

    new_x = _pre_mlp(x_pad, wa, row(b_pre3), wb, row(b_pre1), wa2, wb2, b2c)

    p0 = _sc_agg(new_x, src3, dst3)
    g0 = _gin0(new_x, p0, gin0_w1.T, row(gin0_b1), row(gin0_bng),
               row(gin0_bnb), gin0_w2.T, row(gin0_b2), row(bn0_g), row(bn0_b))

    p1 = _sc_agg(g0, src3, dst3)
    return _gin1_pool(g0, p1, new_x, batch3, gin1_w1.T, row(gin1_b1),
                      row(gin1_bng), row(gin1_bnb), gin1_w2.T, row(gin1_b2),
                      row(bn1_g), row(bn1_b), w_post1.T, row(b_post1),
                      w_post2.T, row(b_post2))


# speedup vs baseline: 13.2257x; 1.0414x over previous
import functools

import jax
import jax.numpy as jnp
from jax import lax
from jax.experimental import pallas as pl
from jax.experimental.pallas import tpu as pltpu
from jax.experimental.pallas import tpu_sc as plsc

N = 10000
E = 320000
D = 128
CFEA = 2
H = 64
NG = 64

BN = 1024
N_PAD = 10240
GRID = N_PAD // BN

NTILES = 32
EPT = 10240
E_PAD = NTILES * EPT
CHUNK = 128
NCH = EPT // CHUNK
NBUF = 3
RPS = N_PAD // 16

_BN_INV = (1.0 + 1e-5) ** -0.5


@functools.cache
def _make_sc_agg():
    mesh = plsc.VectorSubcoreMesh(
        core_axis_name="c", subcore_axis_name="s", num_cores=2, num_subcores=16
    )
    return pl.kernel(
        _sc_agg_body,
        out_type=jax.ShapeDtypeStruct((2, N_PAD, H), jnp.float32),
        mesh=mesh,
        scratch_types=[
            pltpu.VMEM((NCH, CHUNK), jnp.int32),
            pltpu.VMEM((NCH, CHUNK), jnp.int32),
            pltpu.VMEM((NBUF, CHUNK, H), jnp.float32),
            pltpu.VMEM_SHARED((N_PAD, H), jnp.float32),
            pltpu.VMEM_SHARED((N_PAD, H), jnp.float32),
        ] + [pltpu.SemaphoreType.DMA] * (2 * NBUF),
        compiler_params=pltpu.CompilerParams(use_tc_tiling_on_sc=False),
    )


def _sc_agg(h, ei3):
    return _make_sc_agg()(h, ei3)


def _sc_agg_body(h_hbm, ei_hbm, out_hbm, src_v, dst_v, buf, acc,
                 h_sp, g0, g1, g2, s0, s1, s2):
    gsem = (g0, g1, g2)
    ssem = (s0, s1, s2)
    c = lax.axis_index("c")
    s = lax.axis_index("s")
    w = c * 16 + s
    r0s = s * RPS

    cp_h = pltpu.async_copy(h_hbm.at[pl.ds(r0s, RPS)],
                            h_sp.at[pl.ds(r0s, RPS)], g0)
    cp_s = pltpu.async_copy(ei_hbm.at[0, w], src_v, g1)
    cp_d = pltpu.async_copy(ei_hbm.at[1, w], dst_v, g2)

    def _zrow(i, carry):
        for k4 in range(4):
            buf[0, i, pl.ds(k4 * 16, 16)] = jnp.zeros((16,), jnp.float32)
        return carry

    lax.fori_loop(0, CHUNK, _zrow, 0)

    def _crow(j, carry):
        pltpu.sync_copy(buf.at[0],
                        acc.at[pl.ds(r0s + j * CHUNK, CHUNK)])
        return carry

    lax.fori_loop(0, RPS // CHUNK, _crow, 0)
    cp_h.wait()
    cp_s.wait()
    cp_d.wait()
    plsc.subcore_barrier()

    pltpu.async_copy(h_sp.at[src_v.at[0]], buf.at[0], gsem[0])
    pltpu.async_copy(h_sp.at[src_v.at[1]], buf.at[1], gsem[1])

    def _group(gg, carry):
        for b in range(NBUF):
            j = NBUF * gg + b
            bn = (b + 2) % NBUF
            pltpu.make_async_copy(
                h_sp.at[src_v.at[j]], buf.at[b], gsem[b]).wait()
            pltpu.async_copy(buf.at[b], acc.at[dst_v.at[j]], ssem[b],
                             add=True)

            @pl.when(j >= 1)
            def _(j=j, bn=bn):
                pltpu.make_async_copy(
                    buf.at[bn], acc.at[dst_v.at[j]], ssem[bn]).wait()

            pltpu.async_copy(h_sp.at[src_v.at[j + 2]], buf.at[bn], gsem[bn])
        return carry

    lax.fori_loop(0, (NCH - 2) // NBUF, _group, 0)
    pltpu.make_async_copy(
        h_sp.at[src_v.at[NCH - 2]], buf.at[0], gsem[0]).wait()
    pltpu.async_copy(buf.at[0], acc.at[dst_v.at[NCH - 2]], ssem[0], add=True)
    pltpu.make_async_copy(
        h_sp.at[src_v.at[NCH - 1]], buf.at[1], gsem[1]).wait()
    pltpu.async_copy(buf.at[1], acc.at[dst_v.at[NCH - 1]], ssem[1], add=True)
    for b in range(NBUF):
        pltpu.make_async_copy(buf.at[b], acc.at[dst_v.at[0]], ssem[b]).wait()
    plsc.subcore_barrier()

    pltpu.sync_copy(acc.at[pl.ds(r0s, RPS)], out_hbm.at[c, pl.ds(r0s, RPS)])


def _dot_t(a, w):
    return lax.dot_general(a, w, (((1,), (1,)), ((), ())),
                           preferred_element_type=jnp.float32)


def _pre_body(x_ref, w3_ref, b3_ref, w1_ref, b1_ref, w4_ref, b4_ref,
              w2_ref, b2_ref, o_ref):
    x = x_ref[...]
    ha = jnp.maximum(_dot_t(x[:, : D - CFEA], w3_ref[...]) + b3_ref[...], 0.0)
    hb = jnp.maximum(_dot_t(x[:, D - CFEA:], w1_ref[...]) + b1_ref[...], 0.0)
    o_ref[:, : H // 2] = jnp.maximum(
        _dot_t(ha, w4_ref[...]) + b4_ref[...], 0.0)
    o_ref[:, H // 2:] = jnp.maximum(
        _dot_t(hb, w2_ref[...]) + b2_ref[...], 0.0)


def _gin_mlp(h, p_ref, w1_ref, b1_ref, s1_ref, t1_ref, w2_ref, b2_ref,
             s2_ref, t2_ref):
    z = h + p_ref[0] + p_ref[1]
    y = _dot_t(z, w1_ref[...]) + b1_ref[...]
    y = jnp.maximum(y * (s1_ref[...] * _BN_INV) + t1_ref[...], 0.0)
    m = _dot_t(y, w2_ref[...]) + b2_ref[...]
    return m * (s2_ref[...] * _BN_INV) + t2_ref[...]


def _gin0_body(h_ref, p_ref, w1_ref, b1_ref, s1_ref, t1_ref, w2_ref, b2_ref,
               s2_ref, t2_ref, o_ref):
    h = h_ref[...]
    o_ref[...] = _gin_mlp(h, p_ref, w1_ref, b1_ref, s1_ref, t1_ref,
                          w2_ref, b2_ref, s2_ref, t2_ref) + h


def _gin1_pool_body(h_ref, p_ref, r_ref, batch_ref, w1_ref, b1_ref, s1_ref,
                    t1_ref, w2_ref, b2_ref, s2_ref, t2_ref, wp1_ref, bp1_ref,
                    wp2_ref, bp2_ref, o_ref, acc_ref, cnt_ref):
    i = pl.program_id(0)
    h = h_ref[...]
    g = _gin_mlp(h, p_ref, w1_ref, b1_ref, s1_ref, t1_ref,
                 w2_ref, b2_ref, s2_ref, t2_ref) + h + r_ref[...]

    b = batch_ref[0]
    gid = lax.broadcasted_iota(jnp.int32, (NG, BN), 0)
    oh = (gid == b).astype(jnp.float32)
    part = jnp.dot(oh, g, preferred_element_type=jnp.float32)
    csum = jnp.sum(oh, axis=1, keepdims=True)

    @pl.when(i == 0)
    def _():
        acc_ref[...] = part
        cnt_ref[...] = csum

    @pl.when(i > 0)
    def _():
        acc_ref[...] += part
        cnt_ref[...] += csum

    @pl.when(i == GRID - 1)
    def _():
        pooled = acc_ref[...] / jnp.maximum(cnt_ref[...], 1.0)
        u = jnp.maximum(_dot_t(pooled, wp1_ref[...]) + bp1_ref[...], 0.0)
        v = _dot_t(u, wp2_ref[...]) + bp2_ref[...]
        mx = jnp.max(v, axis=1, keepdims=True)
        lse = jnp.log(jnp.sum(jnp.exp(v - mx), axis=1, keepdims=True)) + mx
        o_ref[...] = v - lse


def _full(shape):
    return pl.BlockSpec(shape, lambda i: tuple(0 for _ in shape))


def _pre_mlp(x_pad, w3, b3, w1, b1, w4, b4, w2, b2):
    return pl.pallas_call(
        _pre_body,
        grid=(GRID,),
        in_specs=[
            pl.BlockSpec((BN, D), lambda i: (i, 0)),
            _full((16, D - CFEA)), _full((1, 16)),
            _full((16, CFEA)), _full((1, 16)),
            _full((H // 2, 16)), _full((1, H // 2)),
            _full((H // 2, 16)), _full((1, H // 2)),
        ],
        out_specs=pl.BlockSpec((BN, H), lambda i: (i, 0)),
        out_shape=jax.ShapeDtypeStruct((N_PAD, H), jnp.float32),
    )(x_pad, w3, b3, w1, b1, w4, b4, w2, b2)


def _gin0(h, p, w1, b1, s1, t1, w2, b2, s2, t2):
    return pl.pallas_call(
        _gin0_body,
        grid=(GRID,),
        in_specs=[
            pl.BlockSpec((BN, H), lambda i: (i, 0)),
            pl.BlockSpec((2, BN, H), lambda i: (0, i, 0)),
            _full((H, H)), _full((1, H)), _full((1, H)), _full((1, H)),
            _full((H, H)), _full((1, H)), _full((1, H)), _full((1, H)),
        ],
        out_specs=pl.BlockSpec((BN, H), lambda i: (i, 0)),
        out_shape=jax.ShapeDtypeStruct((N_PAD, H), jnp.float32),
    )(h, p, w1, b1, s1, t1, w2, b2, s2, t2)


def _gin1_pool(h, p, r, batch3, w1, b1, s1, t1, w2, b2, s2, t2,
               wp1, bp1, wp2, bp2):
    return pl.pallas_call(
        _gin1_pool_body,
        grid=(GRID,),
        in_specs=[
            pl.BlockSpec((BN, H), lambda i: (i, 0)),
            pl.BlockSpec((2, BN, H), lambda i: (0, i, 0)),
            pl.BlockSpec((BN, H), lambda i: (i, 0)),
            pl.BlockSpec((1, 1, BN), lambda i: (i, 0, 0)),
            _full((H, H)), _full((1, H)), _full((1, H)), _full((1, H)),
            _full((H, H)), _full((1, H)), _full((1, H)), _full((1, H)),
            _full((16, H)), _full((1, 16)), _full((7, 16)), _full((1, 7)),
        ],
        out_specs=_full((NG, 7)),
        out_shape=jax.ShapeDtypeStruct((NG, 7), jnp.float32),
        scratch_shapes=[
            pltpu.VMEM((NG, H), jnp.float32),
            pltpu.VMEM((NG, 1), jnp.float32),
        ],
    )(h, p, r, batch3, w1, b1, s1, t1, w2, b2, s2, t2, wp1, bp1, wp2, bp2)


def kernel(x, edge_index, batch, w_pre1, b_pre1, w_pre2, b_pre2, w_pre3,
           b_pre3, w_pre4, b_pre4, gin0_w1, gin0_b1, gin0_bng, gin0_bnb,
           gin0_w2, gin0_b2, gin1_w1, gin1_b1, gin1_bng, gin1_bnb, gin1_w2,
           gin1_b2, bn0_g, bn0_b, bn1_g, bn1_b, w_post1, b_post1, w_post2,
           b_post2):
    x_pad = jnp.pad(x, ((0, N_PAD - N), (0, 0)))
    epad = jnp.concatenate(
        [jnp.zeros((1, E_PAD - E), jnp.int32),
         jnp.full((1, E_PAD - E), N_PAD - 1, jnp.int32)])
    ei3 = jnp.concatenate([edge_index, epad], axis=1).reshape(
        2, NTILES, NCH, CHUNK)
    batch3 = jnp.concatenate(
        [batch, jnp.full((N_PAD - N,), NG, jnp.int32)]
    ).reshape(GRID, 1, BN)

    def row(v):
        return v.reshape(1, -1)

    new_x = _pre_mlp(x_pad, w_pre3, row(b_pre3), w_pre1, row(b_pre1),
                     w_pre4, row(b_pre4), w_pre2, row(b_pre2))

    p0 = _sc_agg(new_x, ei3)
    g0 = _gin0(new_x, p0, gin0_w1, row(gin0_b1), row(gin0_bng),
               row(gin0_bnb), gin0_w2, row(gin0_b2), row(bn0_g), row(bn0_b))

    p1 = _sc_agg(g0, ei3)
    return _gin1_pool(g0, p1, new_x, batch3, gin1_w1, row(gin1_b1),
                      row(gin1_bng), row(gin1_bnb), gin1_w2, row(gin1_b2),
                      row(bn1_g), row(bn1_b), w_post1, row(b_post1),
                      w_post2, row(b_post2))


# speedup vs baseline: 13.6617x; 1.0330x over previous
import functools

import jax
import jax.numpy as jnp
from jax import lax
from jax.experimental import pallas as pl
from jax.experimental.pallas import tpu as pltpu
from jax.experimental.pallas import tpu_sc as plsc

N = 10000
E = 320000
D = 128
CFEA = 2
H = 64
NG = 64

BN = 2048
N_PAD = 10240
GRID = N_PAD // BN

NTILES = 32
EPT = 10240
E_PAD = NTILES * EPT
CHUNK = 128
NCH = EPT // CHUNK
NBUF = 3
RPS = N_PAD // 16

_BN_INV = (1.0 + 1e-5) ** -0.5


@functools.cache
def _make_sc_agg():
    mesh = plsc.VectorSubcoreMesh(
        core_axis_name="c", subcore_axis_name="s", num_cores=2, num_subcores=16
    )
    return pl.kernel(
        _sc_agg_body,
        out_type=jax.ShapeDtypeStruct((2, N_PAD, H), jnp.float32),
        mesh=mesh,
        scratch_types=[
            pltpu.VMEM((NCH, CHUNK), jnp.int32),
            pltpu.VMEM((NCH, CHUNK), jnp.int32),
            pltpu.VMEM((NBUF, CHUNK, H), jnp.float32),
            pltpu.VMEM_SHARED((N_PAD, H), jnp.float32),
            pltpu.VMEM_SHARED((N_PAD, H), jnp.float32),
        ] + [pltpu.SemaphoreType.DMA] * (2 * NBUF),
        compiler_params=pltpu.CompilerParams(use_tc_tiling_on_sc=False),
    )


def _sc_agg(h, ei3):
    return _make_sc_agg()(h, ei3)


def _sc_agg_body(h_hbm, ei_hbm, out_hbm, src_v, dst_v, buf, acc,
                 h_sp, g0, g1, g2, s0, s1, s2):
    gsem = (g0, g1, g2)
    ssem = (s0, s1, s2)
    c = lax.axis_index("c")
    s = lax.axis_index("s")
    w = c * 16 + s
    r0s = s * RPS

    cp_h = pltpu.async_copy(h_hbm.at[pl.ds(r0s, RPS)],
                            h_sp.at[pl.ds(r0s, RPS)], g0)
    cp_s = pltpu.async_copy(ei_hbm.at[0, w], src_v, g1)
    cp_d = pltpu.async_copy(ei_hbm.at[1, w], dst_v, g2)

    def _zrow(i, carry):
        for k4 in range(4):
            buf[0, i, pl.ds(k4 * 16, 16)] = jnp.zeros((16,), jnp.float32)
        return carry

    lax.fori_loop(0, CHUNK, _zrow, 0)

    def _crow(j, carry):
        pltpu.sync_copy(buf.at[0],
                        acc.at[pl.ds(r0s + j * CHUNK, CHUNK)])
        return carry

    lax.fori_loop(0, RPS // CHUNK, _crow, 0)
    cp_h.wait()
    cp_s.wait()
    cp_d.wait()
    plsc.subcore_barrier()

    pltpu.async_copy(h_sp.at[src_v.at[0]], buf.at[0], gsem[0])
    pltpu.async_copy(h_sp.at[src_v.at[1]], buf.at[1], gsem[1])

    def _group(gg, carry):
        for b in range(NBUF):
            j = NBUF * gg + b
            bn = (b + 2) % NBUF
            pltpu.make_async_copy(
                h_sp.at[src_v.at[j]], buf.at[b], gsem[b]).wait()
            pltpu.async_copy(buf.at[b], acc.at[dst_v.at[j]], ssem[b],
                             add=True)

            @pl.when(j >= 1)
            def _(j=j, bn=bn):
                pltpu.make_async_copy(
                    buf.at[bn], acc.at[dst_v.at[j]], ssem[bn]).wait()

            pltpu.async_copy(h_sp.at[src_v.at[j + 2]], buf.at[bn], gsem[bn])
        return carry

    lax.fori_loop(0, (NCH - 2) // NBUF, _group, 0)
    pltpu.make_async_copy(
        h_sp.at[src_v.at[NCH - 2]], buf.at[0], gsem[0]).wait()
    pltpu.async_copy(buf.at[0], acc.at[dst_v.at[NCH - 2]], ssem[0], add=True)
    pltpu.make_async_copy(
        h_sp.at[src_v.at[NCH - 1]], buf.at[1], gsem[1]).wait()
    pltpu.async_copy(buf.at[1], acc.at[dst_v.at[NCH - 1]], ssem[1], add=True)
    for b in range(NBUF):
        pltpu.make_async_copy(buf.at[b], acc.at[dst_v.at[0]], ssem[b]).wait()
    plsc.subcore_barrier()

    pltpu.sync_copy(acc.at[pl.ds(r0s, RPS)], out_hbm.at[c, pl.ds(r0s, RPS)])


def _dot_t(a, w):
    return lax.dot_general(a, w, (((1,), (1,)), ((), ())),
                           preferred_element_type=jnp.float32)


def _pre_body(x_ref, w3_ref, b3_ref, w1_ref, b1_ref, w4_ref, b4_ref,
              w2_ref, b2_ref, o_ref):
    x = x_ref[...]
    ha = jnp.maximum(_dot_t(x[:, : D - CFEA], w3_ref[...]) + b3_ref[...], 0.0)
    hb = jnp.maximum(_dot_t(x[:, D - CFEA:], w1_ref[...]) + b1_ref[...], 0.0)
    o_ref[:, : H // 2] = jnp.maximum(
        _dot_t(ha, w4_ref[...]) + b4_ref[...], 0.0)
    o_ref[:, H // 2:] = jnp.maximum(
        _dot_t(hb, w2_ref[...]) + b2_ref[...], 0.0)


def _gin_mlp(h, p_ref, w1_ref, b1_ref, s1_ref, t1_ref, w2_ref, b2_ref,
             s2_ref, t2_ref):
    z = h + p_ref[0] + p_ref[1]
    y = _dot_t(z, w1_ref[...]) + b1_ref[...]
    y = jnp.maximum(y * (s1_ref[...] * _BN_INV) + t1_ref[...], 0.0)
    m = _dot_t(y, w2_ref[...]) + b2_ref[...]
    return m * (s2_ref[...] * _BN_INV) + t2_ref[...]


def _gin0_body(h_ref, p_ref, w1_ref, b1_ref, s1_ref, t1_ref, w2_ref, b2_ref,
               s2_ref, t2_ref, o_ref):
    h = h_ref[...]
    o_ref[...] = _gin_mlp(h, p_ref, w1_ref, b1_ref, s1_ref, t1_ref,
                          w2_ref, b2_ref, s2_ref, t2_ref) + h


def _gin1_pool_body(h_ref, p_ref, r_ref, batch_ref, w1_ref, b1_ref, s1_ref,
                    t1_ref, w2_ref, b2_ref, s2_ref, t2_ref, wp1_ref, bp1_ref,
                    wp2_ref, bp2_ref, o_ref, acc_ref, cnt_ref):
    i = pl.program_id(0)
    h = h_ref[...]
    g = _gin_mlp(h, p_ref, w1_ref, b1_ref, s1_ref, t1_ref,
                 w2_ref, b2_ref, s2_ref, t2_ref) + h + r_ref[...]

    b = batch_ref[0]
    gid = lax.broadcasted_iota(jnp.int32, (NG, BN), 0)
    oh = (gid == b).astype(jnp.float32)
    part = jnp.dot(oh, g, preferred_element_type=jnp.float32)
    csum = jnp.sum(oh, axis=1, keepdims=True)

    @pl.when(i == 0)
    def _():
        acc_ref[...] = part
        cnt_ref[...] = csum

    @pl.when(i > 0)
    def _():
        acc_ref[...] += part
        cnt_ref[...] += csum

    @pl.when(i == GRID - 1)
    def _():
        pooled = acc_ref[...] / jnp.maximum(cnt_ref[...], 1.0)
        u = jnp.maximum(_dot_t(pooled, wp1_ref[...]) + bp1_ref[...], 0.0)
        v = _dot_t(u, wp2_ref[...]) + bp2_ref[...]
        mx = jnp.max(v, axis=1, keepdims=True)
        lse = jnp.log(jnp.sum(jnp.exp(v - mx), axis=1, keepdims=True)) + mx
        o_ref[...] = v - lse


def _full(shape):
    return pl.BlockSpec(shape, lambda i: tuple(0 for _ in shape))


def _pre_mlp(x_pad, w3, b3, w1, b1, w4, b4, w2, b2):
    return pl.pallas_call(
        _pre_body,
        grid=(GRID,),
        in_specs=[
            pl.BlockSpec((BN, D), lambda i: (i, 0)),
            _full((16, D - CFEA)), _full((1, 16)),
            _full((16, CFEA)), _full((1, 16)),
            _full((H // 2, 16)), _full((1, H // 2)),
            _full((H // 2, 16)), _full((1, H // 2)),
        ],
        out_specs=pl.BlockSpec((BN, H), lambda i: (i, 0)),
        out_shape=jax.ShapeDtypeStruct((N_PAD, H), jnp.float32),
    )(x_pad, w3, b3, w1, b1, w4, b4, w2, b2)


def _gin0(h, p, w1, b1, s1, t1, w2, b2, s2, t2):
    return pl.pallas_call(
        _gin0_body,
        grid=(GRID,),
        in_specs=[
            pl.BlockSpec((BN, H), lambda i: (i, 0)),
            pl.BlockSpec((2, BN, H), lambda i: (0, i, 0)),
            _full((H, H)), _full((1, H)), _full((1, H)), _full((1, H)),
            _full((H, H)), _full((1, H)), _full((1, H)), _full((1, H)),
        ],
        out_specs=pl.BlockSpec((BN, H), lambda i: (i, 0)),
        out_shape=jax.ShapeDtypeStruct((N_PAD, H), jnp.float32),
    )(h, p, w1, b1, s1, t1, w2, b2, s2, t2)


def _gin1_pool(h, p, r, batch3, w1, b1, s1, t1, w2, b2, s2, t2,
               wp1, bp1, wp2, bp2):
    return pl.pallas_call(
        _gin1_pool_body,
        grid=(GRID,),
        in_specs=[
            pl.BlockSpec((BN, H), lambda i: (i, 0)),
            pl.BlockSpec((2, BN, H), lambda i: (0, i, 0)),
            pl.BlockSpec((BN, H), lambda i: (i, 0)),
            pl.BlockSpec((1, 1, BN), lambda i: (i, 0, 0)),
            _full((H, H)), _full((1, H)), _full((1, H)), _full((1, H)),
            _full((H, H)), _full((1, H)), _full((1, H)), _full((1, H)),
            _full((16, H)), _full((1, 16)), _full((7, 16)), _full((1, 7)),
        ],
        out_specs=_full((NG, 7)),
        out_shape=jax.ShapeDtypeStruct((NG, 7), jnp.float32),
        scratch_shapes=[
            pltpu.VMEM((NG, H), jnp.float32),
            pltpu.VMEM((NG, 1), jnp.float32),
        ],
    )(h, p, r, batch3, w1, b1, s1, t1, w2, b2, s2, t2, wp1, bp1, wp2, bp2)


def kernel(x, edge_index, batch, w_pre1, b_pre1, w_pre2, b_pre2, w_pre3,
           b_pre3, w_pre4, b_pre4, gin0_w1, gin0_b1, gin0_bng, gin0_bnb,
           gin0_w2, gin0_b2, gin1_w1, gin1_b1, gin1_bng, gin1_bnb, gin1_w2,
           gin1_b2, bn0_g, bn0_b, bn1_g, bn1_b, w_post1, b_post1, w_post2,
           b_post2):
    x_pad = jnp.pad(x, ((0, N_PAD - N), (0, 0)))
    epad = jnp.concatenate(
        [jnp.zeros((1, E_PAD - E), jnp.int32),
         jnp.full((1, E_PAD - E), N_PAD - 1, jnp.int32)])
    ei3 = jnp.concatenate([edge_index, epad], axis=1).reshape(
        2, NTILES, NCH, CHUNK)
    batch3 = jnp.concatenate(
        [batch, jnp.full((N_PAD - N,), NG, jnp.int32)]
    ).reshape(GRID, 1, BN)

    def row(v):
        return v.reshape(1, -1)

    new_x = _pre_mlp(x_pad, w_pre3, row(b_pre3), w_pre1, row(b_pre1),
                     w_pre4, row(b_pre4), w_pre2, row(b_pre2))

    p0 = _sc_agg(new_x, ei3)
    g0 = _gin0(new_x, p0, gin0_w1, row(gin0_b1), row(gin0_bng),
               row(gin0_bnb), gin0_w2, row(gin0_b2), row(bn0_g), row(bn0_b))

    p1 = _sc_agg(g0, ei3)
    return _gin1_pool(g0, p1, new_x, batch3, gin1_w1, row(gin1_b1),
                      row(gin1_bng), row(gin1_bnb), gin1_w2, row(gin1_b2),
                      row(bn1_g), row(bn1_b), w_post1, row(b_post1),
                      w_post2, row(b_post2))


# speedup vs baseline: 13.8937x; 1.0170x over previous
import functools

import jax
import jax.numpy as jnp
from jax import lax
from jax.experimental import pallas as pl
from jax.experimental.pallas import tpu as pltpu
from jax.experimental.pallas import tpu_sc as plsc

N = 10000
E = 320000
D = 128
CFEA = 2
H = 64
NG = 64

BN = 5120
N_PAD = 10240
GRID = N_PAD // BN

NTILES = 32
EPT = 10240
E_PAD = NTILES * EPT
CHUNK = 128
NCH = EPT // CHUNK
NBUF = 3
RPS = N_PAD // 16

_BN_INV = (1.0 + 1e-5) ** -0.5


@functools.cache
def _make_sc_agg():
    mesh = plsc.VectorSubcoreMesh(
        core_axis_name="c", subcore_axis_name="s", num_cores=2, num_subcores=16
    )
    return pl.kernel(
        _sc_agg_body,
        out_type=jax.ShapeDtypeStruct((2, N_PAD, H), jnp.float32),
        mesh=mesh,
        scratch_types=[
            pltpu.VMEM((NCH, CHUNK), jnp.int32),
            pltpu.VMEM((NCH, CHUNK), jnp.int32),
            pltpu.VMEM((NBUF, CHUNK, H), jnp.float32),
            pltpu.VMEM_SHARED((N_PAD, H), jnp.float32),
            pltpu.VMEM_SHARED((N_PAD, H), jnp.float32),
        ] + [pltpu.SemaphoreType.DMA] * (2 * NBUF),
        compiler_params=pltpu.CompilerParams(use_tc_tiling_on_sc=False),
    )


def _sc_agg(h, ei3):
    return _make_sc_agg()(h, ei3)


def _sc_agg_body(h_hbm, ei_hbm, out_hbm, src_v, dst_v, buf, acc,
                 h_sp, g0, g1, g2, s0, s1, s2):
    gsem = (g0, g1, g2)
    ssem = (s0, s1, s2)
    c = lax.axis_index("c")
    s = lax.axis_index("s")
    w = c * 16 + s
    r0s = s * RPS

    cp_h = pltpu.async_copy(h_hbm.at[pl.ds(r0s, RPS)],
                            h_sp.at[pl.ds(r0s, RPS)], g0)
    cp_s = pltpu.async_copy(ei_hbm.at[0, w], src_v, g1)
    cp_d = pltpu.async_copy(ei_hbm.at[1, w], dst_v, g2)

    def _zrow(i, carry):
        for k4 in range(4):
            buf[0, i, pl.ds(k4 * 16, 16)] = jnp.zeros((16,), jnp.float32)
        return carry

    lax.fori_loop(0, CHUNK, _zrow, 0)

    def _crow(j, carry):
        pltpu.sync_copy(buf.at[0],
                        acc.at[pl.ds(r0s + j * CHUNK, CHUNK)])
        return carry

    lax.fori_loop(0, RPS // CHUNK, _crow, 0)
    cp_h.wait()
    cp_s.wait()
    cp_d.wait()
    plsc.subcore_barrier()

    pltpu.async_copy(h_sp.at[src_v.at[0]], buf.at[0], gsem[0])
    pltpu.async_copy(h_sp.at[src_v.at[1]], buf.at[1], gsem[1])

    def _group(gg, carry):
        for b in range(NBUF):
            j = NBUF * gg + b
            bn = (b + 2) % NBUF
            pltpu.make_async_copy(
                h_sp.at[src_v.at[j]], buf.at[b], gsem[b]).wait()
            pltpu.async_copy(buf.at[b], acc.at[dst_v.at[j]], ssem[b],
                             add=True)

            @pl.when(j >= 1)
            def _(j=j, bn=bn):
                pltpu.make_async_copy(
                    buf.at[bn], acc.at[dst_v.at[j]], ssem[bn]).wait()

            pltpu.async_copy(h_sp.at[src_v.at[j + 2]], buf.at[bn], gsem[bn])
        return carry

    lax.fori_loop(0, (NCH - 2) // NBUF, _group, 0)
    pltpu.make_async_copy(
        h_sp.at[src_v.at[NCH - 2]], buf.at[0], gsem[0]).wait()
    pltpu.async_copy(buf.at[0], acc.at[dst_v.at[NCH - 2]], ssem[0], add=True)
    pltpu.make_async_copy(
        h_sp.at[src_v.at[NCH - 1]], buf.at[1], gsem[1]).wait()
    pltpu.async_copy(buf.at[1], acc.at[dst_v.at[NCH - 1]], ssem[1], add=True)
    for b in range(NBUF):
        pltpu.make_async_copy(buf.at[b], acc.at[dst_v.at[0]], ssem[b]).wait()
    plsc.subcore_barrier()

    pltpu.sync_copy(acc.at[pl.ds(r0s, RPS)], out_hbm.at[c, pl.ds(r0s, RPS)])


def _dot_t(a, w):
    return lax.dot_general(a, w, (((1,), (1,)), ((), ())),
                           preferred_element_type=jnp.float32)


def _pre_body(x_ref, w3_ref, b3_ref, w1_ref, b1_ref, w4_ref, b4_ref,
              w2_ref, b2_ref, o_ref):
    x = x_ref[...]
    ha = jnp.maximum(_dot_t(x[:, : D - CFEA], w3_ref[...]) + b3_ref[...], 0.0)
    hb = jnp.maximum(_dot_t(x[:, D - CFEA:], w1_ref[...]) + b1_ref[...], 0.0)
    o_ref[:, : H // 2] = jnp.maximum(
        _dot_t(ha, w4_ref[...]) + b4_ref[...], 0.0)
    o_ref[:, H // 2:] = jnp.maximum(
        _dot_t(hb, w2_ref[...]) + b2_ref[...], 0.0)


def _gin_mlp(h, p_ref, w1_ref, b1_ref, s1_ref, t1_ref, w2_ref, b2_ref,
             s2_ref, t2_ref):
    z = h + p_ref[0] + p_ref[1]
    y = _dot_t(z, w1_ref[...]) + b1_ref[...]
    y = jnp.maximum(y * (s1_ref[...] * _BN_INV) + t1_ref[...], 0.0)
    m = _dot_t(y, w2_ref[...]) + b2_ref[...]
    return m * (s2_ref[...] * _BN_INV) + t2_ref[...]


def _gin0_body(h_ref, p_ref, w1_ref, b1_ref, s1_ref, t1_ref, w2_ref, b2_ref,
               s2_ref, t2_ref, o_ref):
    h = h_ref[...]
    o_ref[...] = _gin_mlp(h, p_ref, w1_ref, b1_ref, s1_ref, t1_ref,
                          w2_ref, b2_ref, s2_ref, t2_ref) + h


def _gin1_pool_body(h_ref, p_ref, r_ref, batch_ref, w1_ref, b1_ref, s1_ref,
                    t1_ref, w2_ref, b2_ref, s2_ref, t2_ref, wp1_ref, bp1_ref,
                    wp2_ref, bp2_ref, o_ref, acc_ref, cnt_ref):
    i = pl.program_id(0)
    h = h_ref[...]
    g = _gin_mlp(h, p_ref, w1_ref, b1_ref, s1_ref, t1_ref,
                 w2_ref, b2_ref, s2_ref, t2_ref) + h + r_ref[...]

    b = batch_ref[0]
    gid = lax.broadcasted_iota(jnp.int32, (NG, BN), 0)
    oh = (gid == b).astype(jnp.float32)
    part = jnp.dot(oh, g, preferred_element_type=jnp.float32)
    csum = jnp.sum(oh, axis=1, keepdims=True)

    @pl.when(i == 0)
    def _():
        acc_ref[...] = part
        cnt_ref[...] = csum

    @pl.when(i > 0)
    def _():
        acc_ref[...] += part
        cnt_ref[...] += csum

    @pl.when(i == GRID - 1)
    def _():
        pooled = acc_ref[...] / jnp.maximum(cnt_ref[...], 1.0)
        u = jnp.maximum(_dot_t(pooled, wp1_ref[...]) + bp1_ref[...], 0.0)
        v = _dot_t(u, wp2_ref[...]) + bp2_ref[...]
        mx = jnp.max(v, axis=1, keepdims=True)
        lse = jnp.log(jnp.sum(jnp.exp(v - mx), axis=1, keepdims=True)) + mx
        o_ref[...] = v - lse


def _full(shape):
    return pl.BlockSpec(shape, lambda i: tuple(0 for _ in shape))


def _pre_mlp(x_pad, w3, b3, w1, b1, w4, b4, w2, b2):
    return pl.pallas_call(
        _pre_body,
        grid=(GRID,),
        in_specs=[
            pl.BlockSpec((BN, D), lambda i: (i, 0)),
            _full((16, D - CFEA)), _full((1, 16)),
            _full((16, CFEA)), _full((1, 16)),
            _full((H // 2, 16)), _full((1, H // 2)),
            _full((H // 2, 16)), _full((1, H // 2)),
        ],
        out_specs=pl.BlockSpec((BN, H), lambda i: (i, 0)),
        out_shape=jax.ShapeDtypeStruct((N_PAD, H), jnp.float32),
    )(x_pad, w3, b3, w1, b1, w4, b4, w2, b2)


def _gin0(h, p, w1, b1, s1, t1, w2, b2, s2, t2):
    return pl.pallas_call(
        _gin0_body,
        grid=(GRID,),
        in_specs=[
            pl.BlockSpec((BN, H), lambda i: (i, 0)),
            pl.BlockSpec((2, BN, H), lambda i: (0, i, 0)),
            _full((H, H)), _full((1, H)), _full((1, H)), _full((1, H)),
            _full((H, H)), _full((1, H)), _full((1, H)), _full((1, H)),
        ],
        out_specs=pl.BlockSpec((BN, H), lambda i: (i, 0)),
        out_shape=jax.ShapeDtypeStruct((N_PAD, H), jnp.float32),
    )(h, p, w1, b1, s1, t1, w2, b2, s2, t2)


def _gin1_pool(h, p, r, batch3, w1, b1, s1, t1, w2, b2, s2, t2,
               wp1, bp1, wp2, bp2):
    return pl.pallas_call(
        _gin1_pool_body,
        grid=(GRID,),
        in_specs=[
            pl.BlockSpec((BN, H), lambda i: (i, 0)),
            pl.BlockSpec((2, BN, H), lambda i: (0, i, 0)),
            pl.BlockSpec((BN, H), lambda i: (i, 0)),
            pl.BlockSpec((1, 1, BN), lambda i: (i, 0, 0)),
            _full((H, H)), _full((1, H)), _full((1, H)), _full((1, H)),
            _full((H, H)), _full((1, H)), _full((1, H)), _full((1, H)),
            _full((16, H)), _full((1, 16)), _full((7, 16)), _full((1, 7)),
        ],
        out_specs=_full((NG, 7)),
        out_shape=jax.ShapeDtypeStruct((NG, 7), jnp.float32),
        scratch_shapes=[
            pltpu.VMEM((NG, H), jnp.float32),
            pltpu.VMEM((NG, 1), jnp.float32),
        ],
    )(h, p, r, batch3, w1, b1, s1, t1, w2, b2, s2, t2, wp1, bp1, wp2, bp2)


def kernel(x, edge_index, batch, w_pre1, b_pre1, w_pre2, b_pre2, w_pre3,
           b_pre3, w_pre4, b_pre4, gin0_w1, gin0_b1, gin0_bng, gin0_bnb,
           gin0_w2, gin0_b2, gin1_w1, gin1_b1, gin1_bng, gin1_bnb, gin1_w2,
           gin1_b2, bn0_g, bn0_b, bn1_g, bn1_b, w_post1, b_post1, w_post2,
           b_post2):
    x_pad = jnp.pad(x, ((0, N_PAD - N), (0, 0)))
    epad = jnp.concatenate(
        [jnp.zeros((1, E_PAD - E), jnp.int32),
         jnp.full((1, E_PAD - E), N_PAD - 1, jnp.int32)])
    ei3 = jnp.concatenate([edge_index, epad], axis=1).reshape(
        2, NTILES, NCH, CHUNK)
    batch3 = jnp.concatenate(
        [batch, jnp.full((N_PAD - N,), NG, jnp.int32)]
    ).reshape(GRID, 1, BN)

    def row(v):
        return v.reshape(1, -1)

    new_x = _pre_mlp(x_pad, w_pre3, row(b_pre3), w_pre1, row(b_pre1),
                     w_pre4, row(b_pre4), w_pre2, row(b_pre2))

    p0 = _sc_agg(new_x, ei3)
    g0 = _gin0(new_x, p0, gin0_w1, row(gin0_b1), row(gin0_bng),
               row(gin0_bnb), gin0_w2, row(gin0_b2), row(bn0_g), row(bn0_b))

    p1 = _sc_agg(g0, ei3)
    return _gin1_pool(g0, p1, new_x, batch3, gin1_w1, row(gin1_b1),
                      row(gin1_bng), row(gin1_bnb), gin1_w2, row(gin1_b2),
                      row(bn1_g), row(bn1_b), w_post1, row(b_post1),
                      w_post2, row(b_post2))
